# baseline (speedup 1.0000x reference)
import functools

import jax
import jax.numpy as jnp
from jax import lax
from jax.experimental import pallas as pl
from jax.experimental.pallas import tpu as pltpu
from jax.experimental.pallas import tpu_sc as plsc

D_MODEL = 2048
MAX_POSITIONS = 512

_NC = 2
_NS = 16
_NW = _NC * _NS

_K = 16
_NCHUNK = 32
_ROWS_PER_TILE = MAX_POSITIONS // _NS


def _sc_gather(ids_hbm, table_hbm, out_hbm, idx_v, rows_v, gsem, ssem):
    wid = lax.axis_index("s") * _NC + lax.axis_index("c")
    base = wid * (_NCHUNK * _K)
    pltpu.sync_copy(ids_hbm.at[wid], idx_v)

    gathers = [None, None]
    scatters = [None, None]

    def start_gather(c):
        b = c % 2
        g = pltpu.async_copy(table_hbm.at[idx_v.at[c]], rows_v.at[b], gsem.at[b])
        gathers[b] = g

    start_gather(0)
    start_gather(1)
    for c in range(_NCHUNK):
        b = c % 2
        gathers[b].wait()
        s = pltpu.async_copy(rows_v.at[b], out_hbm.at[pl.ds(base + c * _K, _K)],
                             ssem.at[b])
        scatters[b] = s
        if c + 2 < _NCHUNK:
            scatters[b].wait()
            start_gather(c + 2)
    scatters[(_NCHUNK - 2) % 2].wait()
    scatters[(_NCHUNK - 1) % 2].wait()


@functools.partial(jax.jit, static_argnames=())
def kernel(branch_ids, pe):
    b, s = branch_ids.shape
    n = b * s
    ids = jnp.clip(branch_ids.astype(jnp.int32), 0, MAX_POSITIONS - 1)
    ids3 = ids.reshape(_NW, _NCHUNK, _K)

    mesh = plsc.VectorSubcoreMesh(core_axis_name="c", subcore_axis_name="s")
    out = pl.kernel(
        _sc_gather,
        out_type=jax.ShapeDtypeStruct((n, D_MODEL), jnp.float32),
        mesh=mesh,
        scratch_types=[
            pltpu.VMEM((_NCHUNK, _K), jnp.int32),
            pltpu.VMEM((2, _K, D_MODEL), jnp.float32),
            pltpu.SemaphoreType.DMA((2,)),
            pltpu.SemaphoreType.DMA((2,)),
        ],
    )(ids3, pe)
    return out.reshape(b, s, D_MODEL)
